# baseline (speedup 1.0000x reference)
import functools

import jax
import jax.numpy as jnp
from jax import lax
from jax.experimental import pallas as pl
from jax.experimental.pallas import tpu as pltpu
from jax.experimental.pallas import tpu_sc as plsc

B = 16384
F = 26
V = 100000
D = 32
DENSE = 13
FD = F * D
FH = F // 2
FDH = FH * D
H1 = 512
H2 = 256

NC = 2
NS = 16
NW = NC * NS
ROWS = B * FH
RPW = ROWS // NW
IDX_MINOR = 128
IDX_ROWS = RPW // IDX_MINOR
G = 13
CHUNK = G * IDX_MINOR
NCHUNK = IDX_ROWS // G


def _sc_gather_half(table_flat, idx3d):
    mesh = plsc.VectorSubcoreMesh(core_axis_name="c", subcore_axis_name="s",
                                  num_cores=NC, num_subcores=NS)

    @functools.partial(
        pl.kernel,
        out_type=jax.ShapeDtypeStruct((ROWS, D), jnp.float32),
        mesh=mesh,
        scratch_types=[
            pltpu.VMEM((IDX_ROWS, IDX_MINOR), jnp.int32),
            pltpu.VMEM((CHUNK, D), jnp.float32),
            pltpu.VMEM((CHUNK, D), jnp.float32),
            pltpu.SemaphoreType.DMA,
            pltpu.SemaphoreType.DMA,
            pltpu.SemaphoreType.DMA,
            pltpu.SemaphoreType.DMA,
        ],
        compiler_params=pltpu.CompilerParams(use_tc_tiling_on_sc=False),
    )
    def gather_kernel(table_hbm, idx_hbm, out_hbm, idx_v, buf0, buf1, gsem0,
                      gsem1, osem0, osem1):
        wid = lax.axis_index("s") * NC + lax.axis_index("c")
        base = wid * RPW
        pltpu.sync_copy(idx_hbm.at[wid], idx_v)

        bufs = (buf0, buf1)
        gsems = (gsem0, gsem1)
        osems = (osem0, osem1)

        def fire_gather(ch, slot):
            for j in range(G):
                pltpu.async_copy(
                    table_hbm.at[idx_v.at[ch * G + j]],
                    bufs[slot].at[pl.ds(j * IDX_MINOR, IDX_MINOR)],
                    gsems[slot])

        def wait_gather(ch, slot):
            for j in range(G):
                pltpu.make_async_copy(
                    table_hbm.at[idx_v.at[ch * G + j]],
                    bufs[slot].at[pl.ds(j * IDX_MINOR, IDX_MINOR)],
                    gsems[slot]).wait()

        def fire_out(ch, slot):
            pltpu.async_copy(bufs[slot],
                             out_hbm.at[pl.ds(base + ch * CHUNK, CHUNK)],
                             osems[slot])

        def wait_out(ch, slot):
            pltpu.make_async_copy(
                bufs[slot], out_hbm.at[pl.ds(base + ch * CHUNK, CHUNK)],
                osems[slot]).wait()

        fire_gather(0, 0)
        for ch in range(NCHUNK):
            slot = ch % 2
            wait_gather(ch, slot)
            if ch + 1 < NCHUNK:
                if ch >= 1:
                    wait_out(ch - 1, (ch - 1) % 2)
                fire_gather(ch + 1, (ch + 1) % 2)
            fire_out(ch, slot)
        wait_out(NCHUNK - 2, (NCHUNK - 2) % 2)
        wait_out(NCHUNK - 1, (NCHUNK - 1) % 2)

    return gather_kernel(table_flat, idx3d)


def _tc_body(dense_ref, e1_ref, e2_ref, w1a_ref, w1b1_ref, w1b2_ref, b1_ref,
             w2_ref, b2_ref, wout_ref, wfmd_ref, wfme1_ref, wfme2_ref,
             bias_ref, out_ref):
    x_d = dense_ref[...]
    x1 = e1_ref[...]
    x2 = e2_ref[...]

    h1 = jnp.dot(x_d, w1a_ref[...], preferred_element_type=jnp.float32)
    h1 = h1 + jnp.dot(x1, w1b1_ref[...], preferred_element_type=jnp.float32)
    h1 = h1 + jnp.dot(x2, w1b2_ref[...], preferred_element_type=jnp.float32)
    h1 = jnp.maximum(h1 + b1_ref[...], 0.0)
    h2 = jnp.dot(h1, w2_ref[...], preferred_element_type=jnp.float32)
    h2 = jnp.maximum(h2 + b2_ref[...], 0.0)
    deep = jnp.sum(h2 * wout_ref[...], axis=1)

    linear = jnp.sum(x_d * wfmd_ref[...], axis=1) + \
        jnp.sum(x1 * wfme1_ref[...], axis=1) + \
        jnp.sum(x2 * wfme2_ref[...], axis=1)

    r = lax.broadcasted_iota(jnp.int32, (FDH, D), 0)
    c = lax.broadcasted_iota(jnp.int32, (FDH, D), 1)
    A = jnp.where(lax.rem(r, D) == c, 1.0, 0.0)
    s = jnp.dot(x1, A, preferred_element_type=jnp.float32)
    s = s + jnp.dot(x2, A, preferred_element_type=jnp.float32)
    sq = jnp.sum(x1 * x1, axis=1) + jnp.sum(x2 * x2, axis=1)
    inter = 0.5 * (jnp.sum(s * s, axis=1) - sq)

    z = linear + inter + deep + bias_ref[0]
    out_ref[...] = jax.nn.sigmoid(z)


def _tc_dense(dense, e1, e2, w1a, w1b1, w1b2, b1r, w2, b2r, woutr, wfmd,
              wfme1, wfme2, biasr, bb):
    grid = (B // bb,)
    return pl.pallas_call(
        _tc_body,
        grid=grid,
        in_specs=[
            pl.BlockSpec((bb, DENSE), lambda i: (i, 0)),
            pl.BlockSpec((bb, FDH), lambda i: (i, 0)),
            pl.BlockSpec((bb, FDH), lambda i: (i, 0)),
            pl.BlockSpec((DENSE, H1), lambda i: (0, 0)),
            pl.BlockSpec((FDH, H1), lambda i: (0, 0)),
            pl.BlockSpec((FDH, H1), lambda i: (0, 0)),
            pl.BlockSpec((1, H1), lambda i: (0, 0)),
            pl.BlockSpec((H1, H2), lambda i: (0, 0)),
            pl.BlockSpec((1, H2), lambda i: (0, 0)),
            pl.BlockSpec((1, H2), lambda i: (0, 0)),
            pl.BlockSpec((1, DENSE), lambda i: (0, 0)),
            pl.BlockSpec((1, FDH), lambda i: (0, 0)),
            pl.BlockSpec((1, FDH), lambda i: (0, 0)),
            pl.BlockSpec((1, 1), lambda i: (0, 0)),
        ],
        out_specs=pl.BlockSpec((bb,), lambda i: (i,)),
        out_shape=jax.ShapeDtypeStruct((B,), jnp.float32),
    )(dense, e1, e2, w1a, w1b1, w1b2, b1r, w2, b2r, woutr, wfmd, wfme1,
      wfme2, biasr)


def _half(sparse_h, tables_h):
    table_flat = tables_h.reshape(FH * V, D)
    idx_flat = sparse_h + jnp.arange(FH, dtype=jnp.int32)[None, :] * V
    idx3d = idx_flat.reshape(NW, IDX_ROWS, IDX_MINOR)
    emb = _sc_gather_half(table_flat, idx3d)
    return emb.reshape(B, FDH)


def kernel(dense, sparse, tables, W_fm, b_fm, W1, b1, W2, b2, Wout, bout):
    e1 = _half(sparse[:, :FH], tables[:FH])
    e2 = _half(sparse[:, FH:], tables[FH:])

    w1a = W1[:DENSE]
    w1b1 = W1[DENSE:DENSE + FDH]
    w1b2 = W1[DENSE + FDH:]
    b1r = b1.reshape(1, H1)
    b2r = b2.reshape(1, H2)
    woutr = Wout.reshape(1, H2)
    wfmd = W_fm[:DENSE].reshape(1, DENSE)
    wfme1 = W_fm[DENSE:DENSE + FDH].reshape(1, FDH)
    wfme2 = W_fm[DENSE + FDH:].reshape(1, FDH)
    biasr = (b_fm + bout).reshape(1, 1)

    return _tc_dense(dense, e1, e2, w1a, w1b1, w1b2, b1r, W2, b2r, woutr,
                     wfmd, wfme1, wfme2, biasr, 512)


# speedup vs baseline: 1.5037x; 1.5037x over previous
import functools

import jax
import jax.numpy as jnp
from jax import lax
from jax.experimental import pallas as pl
from jax.experimental.pallas import tpu as pltpu
from jax.experimental.pallas import tpu_sc as plsc

B = 16384
F = 26
V = 100000
D = 32
DENSE = 13
FD = F * D
H1 = 512
H2 = 256

NC = 2
NS = 16
NW = NC * NS
BPW = B // NW
IDX_MINOR = 128
G = BPW // IDX_MINOR
F2 = F // 2


def _sc_gather(tables, idx3d):
    mesh = plsc.VectorSubcoreMesh(core_axis_name="c", subcore_axis_name="s",
                                  num_cores=NC, num_subcores=NS)

    @functools.partial(
        pl.kernel,
        out_type=jax.ShapeDtypeStruct((B, FD), jnp.float32),
        mesh=mesh,
        scratch_types=[
            pltpu.VMEM((G, IDX_MINOR), jnp.int32),
            pltpu.VMEM((G, IDX_MINOR), jnp.int32),
            pltpu.VMEM((BPW, D), jnp.float32),
            pltpu.VMEM((BPW, D), jnp.float32),
            pltpu.SemaphoreType.DMA,
            pltpu.SemaphoreType.DMA,
            pltpu.SemaphoreType.DMA,
            pltpu.SemaphoreType.DMA,
            pltpu.SemaphoreType.DMA,
            pltpu.SemaphoreType.DMA,
        ],
        compiler_params=pltpu.CompilerParams(use_tc_tiling_on_sc=False),
    )
    def gather_kernel(table_hbm, idx_hbm, out_hbm, idx0, idx1, buf0, buf1,
                      isem0, isem1, gsem0, gsem1, osem0, osem1):
        wid = lax.axis_index("s") * NC + lax.axis_index("c")
        b0 = wid * BPW
        ib = wid * G

        idxs = (idx0, idx1)
        bufs = (buf0, buf1)
        isems = (isem0, isem1)
        gsems = (gsem0, gsem1)
        osems = (osem0, osem1)

        def fire_idx(f, slot):
            pltpu.async_copy(idx_hbm.at[f, pl.ds(ib, G)], idxs[slot],
                             isems[slot])

        def wait_idx(f, slot):
            pltpu.make_async_copy(idx_hbm.at[f, pl.ds(ib, G)], idxs[slot],
                                  isems[slot]).wait()

        def fire_gather(f, slot):
            for j in range(G):
                pltpu.async_copy(
                    table_hbm.at[f].at[idxs[slot].at[j]],
                    bufs[slot].at[pl.ds(j * IDX_MINOR, IDX_MINOR)],
                    gsems[slot])

        def wait_gather(f, slot):
            for j in range(G):
                pltpu.make_async_copy(
                    table_hbm.at[f].at[idxs[slot].at[j]],
                    bufs[slot].at[pl.ds(j * IDX_MINOR, IDX_MINOR)],
                    gsems[slot]).wait()

        def fire_out(f, slot):
            pltpu.async_copy(
                bufs[slot],
                out_hbm.at[pl.ds(b0, BPW), pl.ds(f * D, D)], osems[slot])

        def wait_out(f, slot):
            pltpu.make_async_copy(
                bufs[slot],
                out_hbm.at[pl.ds(b0, BPW), pl.ds(f * D, D)],
                osems[slot]).wait()

        fire_idx(0, 0)
        wait_idx(0, 0)
        fire_gather(0, 0)
        fire_idx(1, 1)

        def body(k, carry):
            fa = 2 * k
            fb = fa + 1
            wait_gather(fa, 0)

            @pl.when(k > 0)
            def _():
                wait_out(fa - 1, 1)

            wait_idx(fb, 1)
            fire_gather(fb, 1)
            fire_out(fa, 0)

            @pl.when(k < F2 - 1)
            def _():
                fire_idx(fa + 2, 0)

            wait_gather(fb, 1)
            wait_out(fa, 0)

            @pl.when(k < F2 - 1)
            def _():
                wait_idx(fa + 2, 0)
                fire_gather(fa + 2, 0)
                fire_idx(fa + 3, 1)

            fire_out(fb, 1)
            return carry

        lax.fori_loop(0, F2, body, 0)
        wait_out(F - 1, 1)

    return gather_kernel(tables, idx3d)


def _tc_body(dense_ref, emb_ref, w1a_ref, w1b_ref, b1_ref, w2_ref, b2_ref,
             wout_ref, wfmd_ref, wfme_ref, bias_ref, out_ref):
    x_d = dense_ref[...]
    x_e = emb_ref[...]
    h1 = jnp.dot(x_d, w1a_ref[...], preferred_element_type=jnp.float32)
    h1 = h1 + jnp.dot(x_e, w1b_ref[...], preferred_element_type=jnp.float32)
    h1 = jnp.maximum(h1 + b1_ref[...], 0.0)
    h2 = jnp.dot(h1, w2_ref[...], preferred_element_type=jnp.float32)
    h2 = jnp.maximum(h2 + b2_ref[...], 0.0)
    deep = jnp.sum(h2 * wout_ref[...], axis=1)

    linear = jnp.sum(x_d * wfmd_ref[...], axis=1) + \
        jnp.sum(x_e * wfme_ref[...], axis=1)

    r = lax.broadcasted_iota(jnp.int32, (FD, D), 0)
    c = lax.broadcasted_iota(jnp.int32, (FD, D), 1)
    A = jnp.where(lax.rem(r, D) == c, 1.0, 0.0)
    s = jnp.dot(x_e, A, preferred_element_type=jnp.float32)
    inter = 0.5 * (jnp.sum(s * s, axis=1) - jnp.sum(x_e * x_e, axis=1))

    z = linear + inter + deep + bias_ref[0]
    out_ref[...] = jax.nn.sigmoid(z)


def _tc_dense(dense, emb2d, w1a, w1b, b1r, w2, b2r, woutr, wfmd, wfme, biasr,
              bb):
    grid = (B // bb,)
    return pl.pallas_call(
        _tc_body,
        grid=grid,
        in_specs=[
            pl.BlockSpec((bb, DENSE), lambda i: (i, 0)),
            pl.BlockSpec((bb, FD), lambda i: (i, 0)),
            pl.BlockSpec((DENSE, H1), lambda i: (0, 0)),
            pl.BlockSpec((FD, H1), lambda i: (0, 0)),
            pl.BlockSpec((1, H1), lambda i: (0, 0)),
            pl.BlockSpec((H1, H2), lambda i: (0, 0)),
            pl.BlockSpec((1, H2), lambda i: (0, 0)),
            pl.BlockSpec((1, H2), lambda i: (0, 0)),
            pl.BlockSpec((1, DENSE), lambda i: (0, 0)),
            pl.BlockSpec((1, FD), lambda i: (0, 0)),
            pl.BlockSpec((1, 1), lambda i: (0, 0)),
        ],
        out_specs=pl.BlockSpec((bb,), lambda i: (i,)),
        out_shape=jax.ShapeDtypeStruct((B,), jnp.float32),
    )(dense, emb2d, w1a, w1b, b1r, w2, b2r, woutr, wfmd, wfme, biasr)


def kernel(dense, sparse, tables, W_fm, b_fm, W1, b1, W2, b2, Wout, bout):
    idx3d = sparse.T.reshape(F, B // IDX_MINOR, IDX_MINOR)

    emb2d = _sc_gather(tables, idx3d)

    w1a = W1[:DENSE]
    w1b = W1[DENSE:]
    b1r = b1.reshape(1, H1)
    b2r = b2.reshape(1, H2)
    woutr = Wout.reshape(1, H2)
    wfmd = W_fm[:DENSE].reshape(1, DENSE)
    wfme = W_fm[DENSE:].reshape(1, FD)
    biasr = (b_fm + bout).reshape(1, 1)

    return _tc_dense(dense, emb2d, w1a, w1b, b1r, W2, b2r, woutr, wfmd, wfme,
                     biasr, 512)
